# Initial kernel scaffold; baseline (speedup 1.0000x reference)
import jax
import jax.numpy as jnp
from jax.experimental import pallas as pl


def kernel(feats, edge_index, W1, b1, Wr1, br1, g1, bt1, W2, b2, Wr2, br2, g2, bt2):
    raise NotImplementedError("write your pallas kernel here")


# speedup vs baseline: 4.4334x; 4.4334x over previous
import functools

import jax
import jax.numpy as jnp
from jax import lax
from jax.experimental import pallas as pl
from jax.experimental.pallas import tpu as pltpu
from jax.experimental.pallas import tpu_sc as plsc

N = 10000
E = 320000
D_IN = 128
H = 64

NW = 32
CHUNK = 128
RPT = 80
EPAD = NW * RPT * CHUNK
NACC = 10112
ROWS_PER_TILE = NACC // 16


def _sc_segment_sum(hw, srcp, dstp, zeros):
    mesh = plsc.VectorSubcoreMesh(core_axis_name="c", subcore_axis_name="s")

    @functools.partial(
        pl.kernel,
        mesh=mesh,
        out_type=jax.ShapeDtypeStruct((2, NACC, H), jnp.float32),
        scratch_types=[
            pltpu.VMEM((RPT, CHUNK), jnp.int32),
            pltpu.VMEM((RPT, CHUNK), jnp.int32),
            pltpu.VMEM((CHUNK, H), jnp.float32),
            pltpu.VMEM_SHARED((NACC, H), jnp.float32),
            pltpu.SemaphoreType.DMA,
        ],
        compiler_params=pltpu.CompilerParams(use_tc_tiling_on_sc=False),
    )
    def k(hw_hbm, srcp_hbm, dstp_hbm, zeros_hbm, out_hbm,
          sidx_v, didx_v, rows_v, acc_sh, sem):
        c = lax.axis_index("c")
        s = lax.axis_index("s")
        wid = s * 2 + c
        zbase = s * ROWS_PER_TILE
        pltpu.sync_copy(zeros_hbm.at[pl.ds(zbase, ROWS_PER_TILE)],
                        acc_sh.at[pl.ds(zbase, ROWS_PER_TILE)])
        ibase = wid * RPT
        pltpu.sync_copy(srcp_hbm.at[pl.ds(ibase, RPT)], sidx_v)
        pltpu.sync_copy(dstp_hbm.at[pl.ds(ibase, RPT)], didx_v)
        plsc.subcore_barrier()

        def body(j, carry):
            pltpu.async_copy(hw_hbm.at[sidx_v.at[j]], rows_v, sem).wait()
            pltpu.sync_copy(rows_v, acc_sh.at[didx_v.at[j]], add=True)
            return carry

        lax.fori_loop(0, RPT, body, 0)
        plsc.subcore_barrier()
        pltpu.sync_copy(acc_sh.at[pl.ds(zbase, ROWS_PER_TILE)],
                        out_hbm.at[c].at[pl.ds(zbase, ROWS_PER_TILE)])

    return k(hw, srcp, dstp, zeros)


def _head_body(h_ref, W_ref, Wr_ref, br_ref, hw_ref, res_ref):
    h = h_ref[...]
    hw_ref[...] = jnp.dot(h, W_ref[...], preferred_element_type=jnp.float32)
    r = jnp.dot(h, Wr_ref[...], preferred_element_type=jnp.float32)
    res_ref[...] = jnp.maximum(r + br_ref[...], 0.0)


def _head(h, W, Wr, br):
    d = h.shape[1]
    return pl.pallas_call(
        _head_body,
        out_shape=(jax.ShapeDtypeStruct((N, H), jnp.float32),
                   jax.ShapeDtypeStruct((N, H), jnp.float32)),
    )(h, W, Wr, br)


def _bn(parts_ref, res_ref, b_ref, g_ref, bt_ref):
    agg = parts_ref[0, :N, :] + parts_ref[1, :N, :]
    new = jnp.maximum(agg + b_ref[...], 0.0) + res_ref[...]
    mean = jnp.mean(new, axis=0, keepdims=True)
    var = jnp.mean((new - mean) ** 2, axis=0, keepdims=True)
    return (new - mean) * lax.rsqrt(var + 1e-5) * g_ref[...] + bt_ref[...]


def _mid_body(parts_ref, res_ref, b_ref, g_ref, bt_ref,
              W2_ref, Wr2_ref, br2_ref, hw2_ref, res2_ref):
    h1 = _bn(parts_ref, res_ref, b_ref, g_ref, bt_ref)
    hw2_ref[...] = jnp.dot(h1, W2_ref[...], preferred_element_type=jnp.float32)
    r = jnp.dot(h1, Wr2_ref[...], preferred_element_type=jnp.float32)
    res2_ref[...] = jnp.maximum(r + br2_ref[...], 0.0)


def _mid(parts, res, b, g, bt, W2, Wr2, br2):
    return pl.pallas_call(
        _mid_body,
        out_shape=(jax.ShapeDtypeStruct((N, H), jnp.float32),
                   jax.ShapeDtypeStruct((N, H), jnp.float32)),
    )(parts, res, b, g, bt, W2, Wr2, br2)


def _tail_body(parts_ref, res_ref, b_ref, g_ref, bt_ref, out_ref):
    out_ref[...] = _bn(parts_ref, res_ref, b_ref, g_ref, bt_ref)


def _tail(parts, res, b, g, bt):
    return pl.pallas_call(
        _tail_body,
        out_shape=jax.ShapeDtypeStruct((N, H), jnp.float32),
    )(parts, res, b, g, bt)


def kernel(feats, edge_index, W1, b1, Wr1, br1, g1, bt1,
           W2, b2, Wr2, br2, g2, bt2):
    src = edge_index[0].astype(jnp.int32)
    dst = edge_index[1].astype(jnp.int32)
    pad = EPAD - E
    srcp = jnp.concatenate([src, jnp.zeros((pad,), jnp.int32)]).reshape(-1, CHUNK)
    dstp = jnp.concatenate([dst, jnp.full((pad,), N, jnp.int32)]).reshape(-1, CHUNK)
    zeros = jnp.zeros((NACC, H), jnp.float32)

    hw1, res1 = _head(feats, W1, Wr1, br1)
    parts1 = _sc_segment_sum(hw1, srcp, dstp, zeros)
    hw2, res2 = _mid(parts1, res1, b1, g1, bt1, W2, Wr2, br2)
    parts2 = _sc_segment_sum(hw2, srcp, dstp, zeros)
    return _tail(parts2, res2, b2, g2, bt2)


# speedup vs baseline: 5.0311x; 1.1348x over previous
import functools

import jax
import jax.numpy as jnp
from jax import lax
from jax.experimental import pallas as pl
from jax.experimental.pallas import tpu as pltpu
from jax.experimental.pallas import tpu_sc as plsc

N = 10000
E = 320000
D_IN = 128
H = 64

NW = 32
CHUNK = 128
RPT = 80
KGRP = 4
NGRP = RPT // KGRP
EPAD = NW * RPT * CHUNK
NACC = 10112
ROWS_PER_TILE = NACC // 16


def _sc_segment_sum(hw, srcp, dstp, zeros):
    mesh = plsc.VectorSubcoreMesh(core_axis_name="c", subcore_axis_name="s")

    @functools.partial(
        pl.kernel,
        mesh=mesh,
        out_type=jax.ShapeDtypeStruct((2, NACC, H), jnp.float32),
        scratch_types=[
            pltpu.VMEM((RPT, CHUNK), jnp.int32),
            pltpu.VMEM((RPT, CHUNK), jnp.int32),
            pltpu.VMEM((2 * KGRP, CHUNK, H), jnp.float32),
            pltpu.VMEM_SHARED((NACC, H), jnp.float32),
            pltpu.SemaphoreType.DMA,
            pltpu.SemaphoreType.DMA,
        ],
        compiler_params=pltpu.CompilerParams(use_tc_tiling_on_sc=False),
    )
    def k(hw_hbm, srcp_hbm, dstp_hbm, zeros_hbm, out_hbm,
          sidx_v, didx_v, rows_v, acc_sh, gsem, ssem):
        c = lax.axis_index("c")
        s = lax.axis_index("s")
        wid = s * 2 + c
        zbase = s * ROWS_PER_TILE
        pltpu.sync_copy(zeros_hbm.at[pl.ds(zbase, ROWS_PER_TILE)],
                        acc_sh.at[pl.ds(zbase, ROWS_PER_TILE)])
        ibase = wid * RPT
        pltpu.sync_copy(srcp_hbm.at[pl.ds(ibase, RPT)], sidx_v)
        pltpu.sync_copy(dstp_hbm.at[pl.ds(ibase, RPT)], didx_v)
        plsc.subcore_barrier()

        def fire_gathers(g, base):
            for b in range(KGRP):
                pltpu.async_copy(hw_hbm.at[sidx_v.at[g * KGRP + b]],
                                 rows_v.at[base + b], gsem)

        def drain_gathers(g, base):
            for b in range(KGRP):
                pltpu.make_async_copy(hw_hbm.at[sidx_v.at[g * KGRP + b]],
                                      rows_v.at[base + b], gsem).wait()

        def fire_scatters(g, base):
            for b in range(KGRP):
                pltpu.async_copy(rows_v.at[base + b],
                                 acc_sh.at[didx_v.at[g * KGRP + b]],
                                 ssem, add=True)

        def drain_scatters(g, base):
            for b in range(KGRP):
                pltpu.make_async_copy(rows_v.at[base + b],
                                      acc_sh.at[didx_v.at[g * KGRP + b]],
                                      ssem).wait()

        fire_gathers(0, 0)

        def body(g, carry):
            base = (g % 2) * KGRP
            obase = ((g + 1) % 2) * KGRP
            drain_gathers(g, base)

            @pl.when(g > 0)
            def _():
                drain_scatters(g - 1, obase)

            @pl.when(g + 1 < NGRP)
            def _():
                fire_gathers(g + 1, obase)

            fire_scatters(g, base)
            return carry

        lax.fori_loop(0, NGRP, body, 0)
        drain_scatters(NGRP - 1, ((NGRP - 1) % 2) * KGRP)
        plsc.subcore_barrier()
        pltpu.sync_copy(acc_sh.at[pl.ds(zbase, ROWS_PER_TILE)],
                        out_hbm.at[c].at[pl.ds(zbase, ROWS_PER_TILE)])

    return k(hw, srcp, dstp, zeros)


def _head_body(h_ref, W_ref, Wr_ref, br_ref, hw_ref, res_ref):
    h = h_ref[...]
    hw_ref[...] = jnp.dot(h, W_ref[...], preferred_element_type=jnp.float32)
    r = jnp.dot(h, Wr_ref[...], preferred_element_type=jnp.float32)
    res_ref[...] = jnp.maximum(r + br_ref[...], 0.0)


def _head(h, W, Wr, br):
    d = h.shape[1]
    return pl.pallas_call(
        _head_body,
        out_shape=(jax.ShapeDtypeStruct((N, H), jnp.float32),
                   jax.ShapeDtypeStruct((N, H), jnp.float32)),
    )(h, W, Wr, br)


def _bn(parts_ref, res_ref, b_ref, g_ref, bt_ref):
    agg = parts_ref[0, :N, :] + parts_ref[1, :N, :]
    new = jnp.maximum(agg + b_ref[...], 0.0) + res_ref[...]
    mean = jnp.mean(new, axis=0, keepdims=True)
    var = jnp.mean((new - mean) ** 2, axis=0, keepdims=True)
    return (new - mean) * lax.rsqrt(var + 1e-5) * g_ref[...] + bt_ref[...]


def _mid_body(parts_ref, res_ref, b_ref, g_ref, bt_ref,
              W2_ref, Wr2_ref, br2_ref, hw2_ref, res2_ref):
    h1 = _bn(parts_ref, res_ref, b_ref, g_ref, bt_ref)
    hw2_ref[...] = jnp.dot(h1, W2_ref[...], preferred_element_type=jnp.float32)
    r = jnp.dot(h1, Wr2_ref[...], preferred_element_type=jnp.float32)
    res2_ref[...] = jnp.maximum(r + br2_ref[...], 0.0)


def _mid(parts, res, b, g, bt, W2, Wr2, br2):
    return pl.pallas_call(
        _mid_body,
        out_shape=(jax.ShapeDtypeStruct((N, H), jnp.float32),
                   jax.ShapeDtypeStruct((N, H), jnp.float32)),
    )(parts, res, b, g, bt, W2, Wr2, br2)


def _tail_body(parts_ref, res_ref, b_ref, g_ref, bt_ref, out_ref):
    out_ref[...] = _bn(parts_ref, res_ref, b_ref, g_ref, bt_ref)


def _tail(parts, res, b, g, bt):
    return pl.pallas_call(
        _tail_body,
        out_shape=jax.ShapeDtypeStruct((N, H), jnp.float32),
    )(parts, res, b, g, bt)


def kernel(feats, edge_index, W1, b1, Wr1, br1, g1, bt1,
           W2, b2, Wr2, br2, g2, bt2):
    src = edge_index[0].astype(jnp.int32)
    dst = edge_index[1].astype(jnp.int32)
    pad = EPAD - E
    srcp = jnp.concatenate([src, jnp.zeros((pad,), jnp.int32)]).reshape(-1, CHUNK)
    dstp = jnp.concatenate([dst, jnp.full((pad,), N, jnp.int32)]).reshape(-1, CHUNK)
    zeros = jnp.zeros((NACC, H), jnp.float32)

    hw1, res1 = _head(feats, W1, Wr1, br1)
    parts1 = _sc_segment_sum(hw1, srcp, dstp, zeros)
    hw2, res2 = _mid(parts1, res1, b1, g1, bt1, W2, Wr2, br2)
    parts2 = _sc_segment_sum(hw2, srcp, dstp, zeros)
    return _tail(parts2, res2, b2, g2, bt2)
